# Initial kernel scaffold; baseline (speedup 1.0000x reference)
import jax
import jax.numpy as jnp
from jax.experimental import pallas as pl


def kernel(x_api, x_file, ei_api_api, ei_file_api, ei_api_file, W_in_api, b_in_api, W_in_file, b_in_file, Wl, bl, Wr, Wc1, bc1, Wc2, bc2):
    raise NotImplementedError("write your pallas kernel here")


# speedup vs baseline: 3.5503x; 3.5503x over previous
import functools

import jax
import jax.numpy as jnp
from jax import lax
from jax.experimental import pallas as pl
from jax.experimental.pallas import tpu as pltpu
from jax.experimental.pallas import tpu_sc as plsc

_H = 64
_HH = 32
_K = 128


def _pad_up(n, m):
    return -(-n // m) * m


def _segsum_sc(table, src, dst, n_src, n_dst):
    e = src.shape[0]
    ep = _pad_up(e, 16 * _K)
    if ep != e:
        pad = ep - e
        src = jnp.concatenate([src, jnp.zeros((pad,), jnp.int32)])
        dst = jnp.concatenate([dst, jnp.full((pad,), n_dst, jnp.int32)])
    np_ = _pad_up(n_dst + 1, 128)
    rpt = np_ // 16
    et = ep // 16
    nck = et // _K
    nfull = rpt // _K
    rem = rpt % _K

    @functools.partial(
        pl.kernel,
        mesh=plsc.VectorSubcoreMesh(core_axis_name="c", subcore_axis_name="s"),
        out_type=jax.ShapeDtypeStruct((2 * np_, _HH), jnp.float32),
        scratch_types=[
            pltpu.VMEM((_K,), jnp.int32),
            pltpu.VMEM((_K,), jnp.int32),
            pltpu.VMEM((_K, _HH), jnp.float32),
            pltpu.VMEM((_K, _HH), jnp.float32),
            pltpu.VMEM_SHARED((np_, _HH), jnp.float32),
            pltpu.SemaphoreType.DMA,
        ],
        compiler_params=pltpu.CompilerParams(use_tc_tiling_on_sc=False),
    )
    def seg_kernel(table_hbm, src_hbm, dst_hbm, out_hbm, sbuf, dbuf, rows, zbuf, acc, sem):
        c = lax.axis_index("c")
        s = lax.axis_index("s")

        def _zrow(i, carry):
            zbuf[i, pl.ds(0, 16)] = jnp.zeros((16,), jnp.float32)
            zbuf[i, pl.ds(16, 16)] = jnp.zeros((16,), jnp.float32)
            return carry

        lax.fori_loop(0, _K, _zrow, 0)

        base_r = s * rpt

        def _zc(j, carry):
            pltpu.sync_copy(zbuf, acc.at[pl.ds(base_r + j * _K, _K)])
            return carry

        lax.fori_loop(0, nfull, _zc, 0)
        if rem:
            pltpu.sync_copy(zbuf.at[pl.ds(0, rem)], acc.at[pl.ds(base_r + nfull * _K, rem)])
        plsc.subcore_barrier()

        coff = c * n_src
        ebase = s * et

        def _chunk(j, carry):
            b = ebase + j * _K
            pltpu.sync_copy(src_hbm.at[pl.ds(b, _K)], sbuf)
            pltpu.sync_copy(dst_hbm.at[pl.ds(b, _K)], dbuf)
            for i in range(_K // 16):
                sl = pl.ds(i * 16, 16)
                sbuf[sl] = sbuf[sl] + coff
            pltpu.async_copy(table_hbm.at[sbuf], rows, sem).wait()
            pltpu.sync_copy(rows, acc.at[dbuf], add=True)
            return carry

        lax.fori_loop(0, nck, _chunk, 0)
        plsc.subcore_barrier()

        obase = c * np_ + base_r

        def _dc(j, carry):
            pltpu.sync_copy(acc.at[pl.ds(base_r + j * _K, _K)],
                            out_hbm.at[pl.ds(obase + j * _K, _K)])
            return carry

        lax.fori_loop(0, nfull, _dc, 0)
        if rem:
            pltpu.sync_copy(acc.at[pl.ds(base_r + nfull * _K, rem)],
                            out_hbm.at[pl.ds(obase + nfull * _K, rem)])

    return seg_kernel(table, src, dst).reshape(2, np_, _HH)


def _counts_sc(dst, n_dst):
    e = dst.shape[0]
    ep = _pad_up(e, 32 * _K)
    if ep != e:
        dst = jnp.concatenate([dst, jnp.full((ep - e,), n_dst, jnp.int32)])
    ncp = _pad_up(n_dst + 1, 128)
    rpt = ncp // 16
    et = ep // 32
    nck = et // _K
    nfull = rpt // _K
    rem = rpt % _K

    @functools.partial(
        pl.kernel,
        mesh=plsc.VectorSubcoreMesh(core_axis_name="c", subcore_axis_name="s"),
        out_type=jax.ShapeDtypeStruct((2 * ncp, 16), jnp.float32),
        scratch_types=[
            pltpu.VMEM((_K,), jnp.int32),
            pltpu.VMEM((_K, 16), jnp.float32),
            pltpu.VMEM((_K, 16), jnp.float32),
            pltpu.VMEM_SHARED((ncp, 16), jnp.float32),
        ],
        compiler_params=pltpu.CompilerParams(use_tc_tiling_on_sc=False),
    )
    def cnt_kernel(dst_hbm, out_hbm, dbuf, ones, zbuf, acc):
        c = lax.axis_index("c")
        s = lax.axis_index("s")

        def _fill(i, carry):
            ones[i, pl.ds(0, 16)] = jnp.full((16,), 1.0, jnp.float32)
            zbuf[i, pl.ds(0, 16)] = jnp.zeros((16,), jnp.float32)
            return carry

        lax.fori_loop(0, _K, _fill, 0)

        base_r = s * rpt

        def _zc(j, carry):
            pltpu.sync_copy(zbuf, acc.at[pl.ds(base_r + j * _K, _K)])
            return carry

        lax.fori_loop(0, nfull, _zc, 0)
        if rem:
            pltpu.sync_copy(zbuf.at[pl.ds(0, rem)], acc.at[pl.ds(base_r + nfull * _K, rem)])
        plsc.subcore_barrier()

        ebase = (c * 16 + s) * et

        def _chunk(j, carry):
            pltpu.sync_copy(dst_hbm.at[pl.ds(ebase + j * _K, _K)], dbuf)
            pltpu.sync_copy(ones, acc.at[dbuf], add=True)
            return carry

        lax.fori_loop(0, nck, _chunk, 0)
        plsc.subcore_barrier()

        obase = c * ncp + base_r

        def _dc(j, carry):
            pltpu.sync_copy(acc.at[pl.ds(base_r + j * _K, _K)],
                            out_hbm.at[pl.ds(obase + j * _K, _K)])
            return carry

        lax.fori_loop(0, nfull, _dc, 0)
        if rem:
            pltpu.sync_copy(acc.at[pl.ds(base_r + nfull * _K, rem)],
                            out_hbm.at[pl.ds(obase + nfull * _K, rem)])

    return cnt_kernel(dst).reshape(2, ncp, 16)


def _proj_tc(x, w, b):
    n, f = x.shape
    r = 1000
    assert n % r == 0

    def body(x_ref, w_ref, b_ref, o_ref):
        h = jnp.dot(x_ref[...], w_ref[...], preferred_element_type=jnp.float32)
        h = jnp.maximum(h + b_ref[...], 0.0)
        o_ref[0] = h[:, :_HH]
        o_ref[1] = h[:, _HH:]

    return pl.pallas_call(
        body,
        grid=(n // r,),
        in_specs=[
            pl.BlockSpec((r, f), lambda i: (i, 0)),
            pl.BlockSpec((f, _H), lambda i: (0, 0)),
            pl.BlockSpec((1, _H), lambda i: (0, 0)),
        ],
        out_specs=pl.BlockSpec((2, r, _HH), lambda i: (0, i, 0)),
        out_shape=jax.ShapeDtypeStruct((2, n, _HH), jnp.float32),
    )(x, w, b.reshape(1, _H))


def _api_layer_tc(seg_aa, cnt_aa, seg_fa, cnt_fa, h, wl0, wl1, wr, bsum):
    n = h.shape[1]
    r = 1000
    assert n % r == 0
    grid = n // r

    def body(saa, caa, sfa, cfa, h_ref, w0, w1, wr_ref, b_ref, o_ref, sum_ref):
        inv_aa = 1.0 / jnp.maximum(caa[0, :, 0] + caa[1, :, 0], 1.0)
        inv_fa = 1.0 / jnp.maximum(cfa[0, :, 0] + cfa[1, :, 0], 1.0)
        w0v = w0[...]
        w1v = w1[...]
        wrv = wr_ref[...]
        o = jnp.dot(saa[0] * inv_aa[:, None], w0v[:_HH], preferred_element_type=jnp.float32)
        o += jnp.dot(saa[1] * inv_aa[:, None], w0v[_HH:], preferred_element_type=jnp.float32)
        o += jnp.dot(sfa[0] * inv_fa[:, None], w1v[:_HH], preferred_element_type=jnp.float32)
        o += jnp.dot(sfa[1] * inv_fa[:, None], w1v[_HH:], preferred_element_type=jnp.float32)
        o += jnp.dot(h_ref[0], wrv[:_HH], preferred_element_type=jnp.float32)
        o += jnp.dot(h_ref[1], wrv[_HH:], preferred_element_type=jnp.float32)
        o = jnp.maximum(o + b_ref[...], 0.0)
        o_ref[0] = o[:, :_HH]
        o_ref[1] = o[:, _HH:]

        @pl.when(pl.program_id(0) == 0)
        def _():
            sum_ref[...] = jnp.zeros_like(sum_ref)

        sum_ref[...] += jnp.sum(o, axis=0, keepdims=True)

    return pl.pallas_call(
        body,
        grid=(grid,),
        in_specs=[
            pl.BlockSpec((2, r, _HH), lambda i: (0, i, 0)),
            pl.BlockSpec((2, r, 16), lambda i: (0, i, 0)),
            pl.BlockSpec((2, r, _HH), lambda i: (0, i, 0)),
            pl.BlockSpec((2, r, 16), lambda i: (0, i, 0)),
            pl.BlockSpec((2, r, _HH), lambda i: (0, i, 0)),
            pl.BlockSpec((_H, _H), lambda i: (0, 0)),
            pl.BlockSpec((_H, _H), lambda i: (0, 0)),
            pl.BlockSpec((_H, _H), lambda i: (0, 0)),
            pl.BlockSpec((1, _H), lambda i: (0, 0)),
        ],
        out_specs=[
            pl.BlockSpec((2, r, _HH), lambda i: (0, i, 0)),
            pl.BlockSpec((1, _H), lambda i: (0, 0)),
        ],
        out_shape=[
            jax.ShapeDtypeStruct((2, n, _HH), jnp.float32),
            jax.ShapeDtypeStruct((1, _H), jnp.float32),
        ],
    )(seg_aa, cnt_aa, seg_fa, cnt_fa, h, wl0, wl1, wr, bsum)


def _file_layer_tc(seg_af, cnt_af, h, wl2, wr2, b2):
    n = h.shape[1]
    r = 1000
    assert n % r == 0
    grid = n // r

    def body(saf, caf, h_ref, w2, wr_ref, b_ref, o_ref, sum_ref):
        inv = 1.0 / jnp.maximum(caf[0, :, 0] + caf[1, :, 0], 1.0)
        w2v = w2[...]
        wrv = wr_ref[...]
        o = jnp.dot(saf[0] * inv[:, None], w2v[:_HH], preferred_element_type=jnp.float32)
        o += jnp.dot(saf[1] * inv[:, None], w2v[_HH:], preferred_element_type=jnp.float32)
        o += jnp.dot(h_ref[0], wrv[:_HH], preferred_element_type=jnp.float32)
        o += jnp.dot(h_ref[1], wrv[_HH:], preferred_element_type=jnp.float32)
        o = jnp.maximum(o + b_ref[...], 0.0)
        o_ref[0] = o[:, :_HH]
        o_ref[1] = o[:, _HH:]

        @pl.when(pl.program_id(0) == 0)
        def _():
            sum_ref[...] = jnp.zeros_like(sum_ref)

        sum_ref[...] += jnp.sum(o, axis=0, keepdims=True)

    return pl.pallas_call(
        body,
        grid=(grid,),
        in_specs=[
            pl.BlockSpec((2, r, _HH), lambda i: (0, i, 0)),
            pl.BlockSpec((2, r, 16), lambda i: (0, i, 0)),
            pl.BlockSpec((2, r, _HH), lambda i: (0, i, 0)),
            pl.BlockSpec((_H, _H), lambda i: (0, 0)),
            pl.BlockSpec((_H, _H), lambda i: (0, 0)),
            pl.BlockSpec((1, _H), lambda i: (0, 0)),
        ],
        out_specs=[
            pl.BlockSpec((2, r, _HH), lambda i: (0, i, 0)),
            pl.BlockSpec((1, _H), lambda i: (0, 0)),
        ],
        out_shape=[
            jax.ShapeDtypeStruct((2, n, _HH), jnp.float32),
            jax.ShapeDtypeStruct((1, _H), jnp.float32),
        ],
    )(seg_af, cnt_af, h, wl2, wr2, b2)


def _head_tc(sum_api, sum_file, n_api, n_file, wc1, bc1, wc2, bc2):
    def body(sa, sf, w1, b1, w2, b2, o_ref):
        w1v = w1[...]
        hid = jnp.dot(sa[...] * (1.0 / n_api), w1v[:_H], preferred_element_type=jnp.float32)
        hid += jnp.dot(sf[...] * (1.0 / n_file), w1v[_H:], preferred_element_type=jnp.float32)
        hid = jnp.maximum(hid + b1[...], 0.0)
        o_ref[...] = jnp.dot(hid, w2[...], preferred_element_type=jnp.float32) + b2[...]

    return pl.pallas_call(
        body,
        out_shape=jax.ShapeDtypeStruct((1, 2), jnp.float32),
    )(sum_api, sum_file, wc1, bc1.reshape(1, _H), wc2, bc2.reshape(1, 2))


def _chain(dep, arr):
    return arr + (dep * 0.0).astype(arr.dtype)


def kernel(x_api, x_file, ei_api_api, ei_file_api, ei_api_file,
           W_in_api, b_in_api, W_in_file, b_in_file,
           Wl, bl, Wr, Wc1, bc1, Wc2, bc2):
    n_api = x_api.shape[0]
    n_file = x_file.shape[0]

    h_api = _proj_tc(x_api, W_in_api, b_in_api)
    h_file = _proj_tc(x_file, W_in_file, b_in_file)

    cnt_aa = _counts_sc(ei_api_api[1], n_api)
    cnt_fa = _counts_sc(_chain(cnt_aa[0, 0, 0], ei_file_api[1]), n_api)
    cnt_af = _counts_sc(_chain(cnt_fa[0, 0, 0], ei_api_file[1]), n_file)
    dep = cnt_af[0, 0, 0]

    sum_api = sum_file = None
    for l in range(2):
        seg_aa = _segsum_sc(h_api.reshape(2 * n_api, _HH),
                            ei_api_api[0], _chain(dep, ei_api_api[1]),
                            n_api, n_api)
        seg_fa = _segsum_sc(h_file.reshape(2 * n_file, _HH),
                            ei_file_api[0], _chain(seg_aa[0, 0, 0], ei_file_api[1]),
                            n_file, n_api)
        seg_af = _segsum_sc(h_api.reshape(2 * n_api, _HH),
                            ei_api_file[0], _chain(seg_fa[0, 0, 0], ei_api_file[1]),
                            n_api, n_file)
        dep = seg_af[0, 0, 0]
        h_api_n, sum_api = _api_layer_tc(
            seg_aa, cnt_aa, seg_fa, cnt_fa, h_api,
            Wl[l, 0], Wl[l, 1], Wr[l, 0] + Wr[l, 1],
            (bl[l, 0] + bl[l, 1]).reshape(1, _H))
        h_file_n, sum_file = _file_layer_tc(
            seg_af, cnt_af, h_file, Wl[l, 2], Wr[l, 2], bl[l, 2].reshape(1, _H))
        h_api, h_file = h_api_n, h_file_n

    return _head_tc(sum_api, sum_file, n_api, n_file, Wc1, bc1, Wc2, bc2)
